# Initial kernel scaffold; baseline (speedup 1.0000x reference)
import jax
import jax.numpy as jnp
from jax.experimental import pallas as pl


def kernel(z, codebook):
    raise NotImplementedError("write your pallas kernel here")


# speedup vs baseline: 1.2516x; 1.2516x over previous
import functools

import jax
import jax.numpy as jnp
from jax import lax
from jax.experimental import pallas as pl
from jax.experimental.pallas import tpu as pltpu
from jax.experimental.pallas import tpu_sc as plsc

_BM = 512
_WINDOWS = (0, 4096, 8192)


def _dist_argmin_body(zn_ref, cbn_ref, a_ref, cbt_ref, idx_ref, loss_ref):
    m = pl.program_id(0)
    K = cbt_ref.shape[1]
    mm = lax.dot_general(
        a_ref[...], cbt_ref[...],
        (((1,), (0,)), ((), ())),
        preferred_element_type=jnp.float32,
    )
    dist = (zn_ref[...] + cbn_ref[...]) - mm

    acc_v = None
    acc_i = None
    for w in range(len(_WINDOWS) - 1):
        lo, hi = _WINDOWS[w], _WINDOWS[w + 1]
        dw = dist[:, lo:hi]
        kidx = lax.broadcasted_iota(jnp.int32, dw.shape, 1) + lo
        cv = jnp.min(dw, axis=1, keepdims=True)
        ci = jnp.min(jnp.where(dw == cv, kidx, K), axis=1, keepdims=True)
        if acc_v is None:
            acc_v, acc_i = cv, ci
        else:
            sw = cv < acc_v
            acc_i = jnp.where(sw, ci, acc_i)
            acc_v = jnp.where(sw, cv, acc_v)
        acc_v = acc_v.astype(jnp.bfloat16).astype(jnp.float32)

    idx_ref[...] = acc_i
    part = jnp.sum(acc_v, axis=(0, 1), keepdims=True)

    @pl.when(m == 0)
    def _():
        loss_ref[...] = part

    @pl.when(m != 0)
    def _():
        loss_ref[...] = loss_ref[...] + part


def _dist_argmin(zn, cbn, a_bf, cbt_bf):
    M, D = a_bf.shape
    K = cbt_bf.shape[1]
    return pl.pallas_call(
        _dist_argmin_body,
        grid=(M // _BM,),
        in_specs=[
            pl.BlockSpec((_BM, 1), lambda m: (m, 0)),
            pl.BlockSpec((1, K), lambda m: (0, 0)),
            pl.BlockSpec((_BM, D), lambda m: (m, 0)),
            pl.BlockSpec((D, K), lambda m: (0, 0)),
        ],
        out_specs=[
            pl.BlockSpec((_BM, 1), lambda m: (m, 0)),
            pl.BlockSpec((1, 1), lambda m: (0, 0)),
        ],
        out_shape=[
            jax.ShapeDtypeStruct((M, 1), jnp.int32),
            jax.ShapeDtypeStruct((1, 1), jnp.float32),
        ],
    )(zn, cbn, a_bf, cbt_bf)


@functools.lru_cache(maxsize=None)
def _make_gather(K, D, M):
    info = plsc.get_sparse_core_info()
    NC, NS = info.num_cores, info.num_subcores
    NW = NC * NS
    b_per_w = M // NW
    CH = 128
    nch = b_per_w // CH
    mesh = plsc.VectorSubcoreMesh(core_axis_name="c", subcore_axis_name="s")

    @functools.partial(
        pl.kernel,
        mesh=mesh,
        out_type=jax.ShapeDtypeStruct((M, D), jnp.float32),
        scratch_types=[
            pltpu.VMEM((CH,), jnp.int32),
            pltpu.VMEM((CH, D), jnp.float32),
            pltpu.VMEM((CH,), jnp.int32),
            pltpu.VMEM((CH, D), jnp.float32),
            pltpu.SemaphoreType.DMA,
            pltpu.SemaphoreType.DMA,
        ],
    )
    def gk(table_hbm, idx_hbm, out_hbm, idx_a, rows_a, idx_b, rows_b, sem_a, sem_b):
        wid = lax.axis_index("s") * NC + lax.axis_index("c")
        base = wid * b_per_w
        ibuf, rbuf, sem = (idx_a, idx_b), (rows_a, rows_b), (sem_a, sem_b)
        cps = [None, None]
        for c in range(nch):
            b = c & 1
            pltpu.sync_copy(idx_hbm.at[pl.ds(base + c * CH, CH)], ibuf[b])
            cps[b] = pltpu.async_copy(table_hbm.at[ibuf[b]], rbuf[b], sem[b])
            if c >= 1:
                pb = (c - 1) & 1
                cps[pb].wait()
                pltpu.sync_copy(rbuf[pb], out_hbm.at[pl.ds(base + (c - 1) * CH, CH)])
        lb = (nch - 1) & 1
        cps[lb].wait()
        pltpu.sync_copy(rbuf[lb], out_hbm.at[pl.ds(base + (nch - 1) * CH, CH)])

    return gk


def kernel(z, codebook):
    B, N, D = z.shape
    K = codebook.shape[0]
    M = B * N
    flat = z.reshape(-1, D)
    zn = jnp.sum(flat ** 2, axis=1, keepdims=True)
    cbn = jnp.sum(codebook ** 2, axis=1)[None, :]
    a_bf = (flat * 2.0).astype(jnp.bfloat16)
    cbt_bf = codebook.T.astype(jnp.bfloat16)
    idx2, loss_sum = _dist_argmin(zn, cbn, a_bf, cbt_bf)
    idx = idx2.reshape(-1)
    quant = _make_gather(K, D, M)(codebook, idx)
    loss = 1.25 * loss_sum[0, 0] / (B * N * D)
    return quant.reshape(z.shape), loss, idx.reshape(B, N)


# speedup vs baseline: 1.2521x; 1.0004x over previous
import functools

import jax
import jax.numpy as jnp
from jax import lax
from jax.experimental import pallas as pl
from jax.experimental.pallas import tpu as pltpu
from jax.experimental.pallas import tpu_sc as plsc

_BM = 512
_WINDOWS = (0, 4096, 8192)


def _dist_argmin_body(zn_ref, cbn_ref, a_ref, cbt_ref, kiota_ref, idx_ref, loss_ref):
    m = pl.program_id(0)
    K = cbt_ref.shape[1]
    mm = lax.dot_general(
        a_ref[...], cbt_ref[...],
        (((1,), (0,)), ((), ())),
        preferred_element_type=jnp.float32,
    )
    dist = (zn_ref[...] + cbn_ref[...]) - mm

    acc_v = None
    acc_i = None
    for w in range(len(_WINDOWS) - 1):
        lo, hi = _WINDOWS[w], _WINDOWS[w + 1]
        dw = dist[:, lo:hi]
        kidx = kiota_ref[...][:, lo:hi]
        cv = jnp.min(dw, axis=1, keepdims=True)
        ci = jnp.min(jnp.where(dw == cv, kidx, K), axis=1, keepdims=True)
        if acc_v is None:
            acc_v, acc_i = cv, ci
        else:
            sw = cv < acc_v
            acc_i = jnp.where(sw, ci, acc_i)
            acc_v = jnp.where(sw, cv, acc_v)
        acc_v = acc_v.astype(jnp.bfloat16).astype(jnp.float32)

    idx_ref[...] = acc_i
    part = jnp.sum(acc_v, axis=(0, 1), keepdims=True)

    @pl.when(m == 0)
    def _():
        loss_ref[...] = part

    @pl.when(m != 0)
    def _():
        loss_ref[...] = loss_ref[...] + part


def _dist_argmin(zn, cbn, a_bf, cbt_bf):
    M, D = a_bf.shape
    K = cbt_bf.shape[1]
    kiota = jnp.arange(K, dtype=jnp.int32)[None, :]
    return pl.pallas_call(
        _dist_argmin_body,
        grid=(M // _BM,),
        in_specs=[
            pl.BlockSpec((_BM, 1), lambda m: (m, 0)),
            pl.BlockSpec((1, K), lambda m: (0, 0)),
            pl.BlockSpec((_BM, D), lambda m: (m, 0)),
            pl.BlockSpec((D, K), lambda m: (0, 0)),
            pl.BlockSpec((1, K), lambda m: (0, 0)),
        ],
        out_specs=[
            pl.BlockSpec((_BM, 1), lambda m: (m, 0)),
            pl.BlockSpec((1, 1), lambda m: (0, 0)),
        ],
        out_shape=[
            jax.ShapeDtypeStruct((M, 1), jnp.int32),
            jax.ShapeDtypeStruct((1, 1), jnp.float32),
        ],
    )(zn, cbn, a_bf, cbt_bf, kiota)


@functools.lru_cache(maxsize=None)
def _make_gather(K, D, M):
    info = plsc.get_sparse_core_info()
    NC, NS = info.num_cores, info.num_subcores
    NW = NC * NS
    b_per_w = M // NW
    CH = 128
    nch = b_per_w // CH
    mesh = plsc.VectorSubcoreMesh(core_axis_name="c", subcore_axis_name="s")

    @functools.partial(
        pl.kernel,
        mesh=mesh,
        out_type=jax.ShapeDtypeStruct((M, D), jnp.float32),
        scratch_types=[
            pltpu.VMEM((CH,), jnp.int32),
            pltpu.VMEM((CH, D), jnp.float32),
            pltpu.VMEM((CH,), jnp.int32),
            pltpu.VMEM((CH, D), jnp.float32),
            pltpu.SemaphoreType.DMA,
            pltpu.SemaphoreType.DMA,
        ],
    )
    def gk(table_hbm, idx_hbm, out_hbm, idx_a, rows_a, idx_b, rows_b, sem_a, sem_b):
        wid = lax.axis_index("s") * NC + lax.axis_index("c")
        base = wid * b_per_w
        ibuf, rbuf, sem = (idx_a, idx_b), (rows_a, rows_b), (sem_a, sem_b)
        cps = [None, None]
        for c in range(nch):
            b = c & 1
            pltpu.sync_copy(idx_hbm.at[pl.ds(base + c * CH, CH)], ibuf[b])
            cps[b] = pltpu.async_copy(table_hbm.at[ibuf[b]], rbuf[b], sem[b])
            if c >= 1:
                pb = (c - 1) & 1
                cps[pb].wait()
                pltpu.sync_copy(rbuf[pb], out_hbm.at[pl.ds(base + (c - 1) * CH, CH)])
        lb = (nch - 1) & 1
        cps[lb].wait()
        pltpu.sync_copy(rbuf[lb], out_hbm.at[pl.ds(base + (nch - 1) * CH, CH)])

    return gk


def kernel(z, codebook):
    B, N, D = z.shape
    K = codebook.shape[0]
    M = B * N
    flat = z.reshape(-1, D)
    zn = jnp.sum(flat ** 2, axis=1, keepdims=True)
    cbn = jnp.sum(codebook ** 2, axis=1)[None, :]
    a_bf = (flat * 2.0).astype(jnp.bfloat16)
    cbt_bf = codebook.T.astype(jnp.bfloat16)
    idx2, loss_sum = _dist_argmin(zn, cbn, a_bf, cbt_bf)
    idx = idx2.reshape(-1)
    quant = _make_gather(K, D, M)(codebook, idx)
    loss = 1.25 * loss_sum[0, 0] / (B * N * D)
    return quant.reshape(z.shape), loss, idx.reshape(B, N)


# speedup vs baseline: 1.2871x; 1.0280x over previous
import functools

import jax
import jax.numpy as jnp
from jax import lax
from jax.experimental import pallas as pl
from jax.experimental.pallas import tpu as pltpu
from jax.experimental.pallas import tpu_sc as plsc

_BM = 1024
_WINDOWS = (0, 4096, 8192)


def _dist_argmin_body(zn_ref, cbn_ref, a_ref, cbt_ref, kiota_ref, idx_ref, loss_ref):
    m = pl.program_id(0)
    K = cbt_ref.shape[1]
    mm = lax.dot_general(
        a_ref[...], cbt_ref[...],
        (((1,), (0,)), ((), ())),
        preferred_element_type=jnp.float32,
    )
    dist = (zn_ref[...] + cbn_ref[...]) - mm

    acc_v = None
    acc_i = None
    for w in range(len(_WINDOWS) - 1):
        lo, hi = _WINDOWS[w], _WINDOWS[w + 1]
        dw = dist[:, lo:hi]
        kidx = kiota_ref[...][:, lo:hi]
        cv = jnp.min(dw, axis=1, keepdims=True)
        ci = jnp.min(jnp.where(dw == cv, kidx, K), axis=1, keepdims=True)
        if acc_v is None:
            acc_v, acc_i = cv, ci
        else:
            sw = cv < acc_v
            acc_i = jnp.where(sw, ci, acc_i)
            acc_v = jnp.where(sw, cv, acc_v)
        acc_v = acc_v.astype(jnp.bfloat16).astype(jnp.float32)

    idx_ref[...] = acc_i
    part = jnp.sum(acc_v, axis=(0, 1), keepdims=True)

    @pl.when(m == 0)
    def _():
        loss_ref[...] = part

    @pl.when(m != 0)
    def _():
        loss_ref[...] = loss_ref[...] + part


def _dist_argmin(zn, cbn, a_bf, cbt_bf):
    M, D = a_bf.shape
    K = cbt_bf.shape[1]
    kiota = jnp.arange(K, dtype=jnp.int32)[None, :]
    return pl.pallas_call(
        _dist_argmin_body,
        grid=(M // _BM,),
        in_specs=[
            pl.BlockSpec((_BM, 1), lambda m: (m, 0)),
            pl.BlockSpec((1, K), lambda m: (0, 0)),
            pl.BlockSpec((_BM, D), lambda m: (m, 0)),
            pl.BlockSpec((D, K), lambda m: (0, 0)),
            pl.BlockSpec((1, K), lambda m: (0, 0)),
        ],
        out_specs=[
            pl.BlockSpec((_BM, 1), lambda m: (m, 0)),
            pl.BlockSpec((1, 1), lambda m: (0, 0)),
        ],
        out_shape=[
            jax.ShapeDtypeStruct((M, 1), jnp.int32),
            jax.ShapeDtypeStruct((1, 1), jnp.float32),
        ],
    )(zn, cbn, a_bf, cbt_bf, kiota)


@functools.lru_cache(maxsize=None)
def _make_gather(K, D, M):
    info = plsc.get_sparse_core_info()
    NC, NS = info.num_cores, info.num_subcores
    NW = NC * NS
    b_per_w = M // NW
    CH = 128
    nch = b_per_w // CH
    mesh = plsc.VectorSubcoreMesh(core_axis_name="c", subcore_axis_name="s")

    @functools.partial(
        pl.kernel,
        mesh=mesh,
        out_type=jax.ShapeDtypeStruct((M, D), jnp.float32),
        scratch_types=[
            pltpu.VMEM((CH,), jnp.int32),
            pltpu.VMEM((CH, D), jnp.float32),
            pltpu.VMEM((CH,), jnp.int32),
            pltpu.VMEM((CH, D), jnp.float32),
            pltpu.SemaphoreType.DMA,
            pltpu.SemaphoreType.DMA,
        ],
    )
    def gk(table_hbm, idx_hbm, out_hbm, idx_a, rows_a, idx_b, rows_b, sem_a, sem_b):
        wid = lax.axis_index("s") * NC + lax.axis_index("c")
        base = wid * b_per_w
        ibuf, rbuf, sem = (idx_a, idx_b), (rows_a, rows_b), (sem_a, sem_b)
        cps = [None, None]
        for c in range(nch):
            b = c & 1
            pltpu.sync_copy(idx_hbm.at[pl.ds(base + c * CH, CH)], ibuf[b])
            cps[b] = pltpu.async_copy(table_hbm.at[ibuf[b]], rbuf[b], sem[b])
            if c >= 1:
                pb = (c - 1) & 1
                cps[pb].wait()
                pltpu.sync_copy(rbuf[pb], out_hbm.at[pl.ds(base + (c - 1) * CH, CH)])
        lb = (nch - 1) & 1
        cps[lb].wait()
        pltpu.sync_copy(rbuf[lb], out_hbm.at[pl.ds(base + (nch - 1) * CH, CH)])

    return gk


def kernel(z, codebook):
    B, N, D = z.shape
    K = codebook.shape[0]
    M = B * N
    flat = z.reshape(-1, D)
    zn = jnp.sum(flat ** 2, axis=1, keepdims=True)
    cbn = jnp.sum(codebook ** 2, axis=1)[None, :]
    a_bf = (flat * 2.0).astype(jnp.bfloat16)
    cbt_bf = codebook.T.astype(jnp.bfloat16)
    idx2, loss_sum = _dist_argmin(zn, cbn, a_bf, cbt_bf)
    idx = idx2.reshape(-1)
    quant = _make_gather(K, D, M)(codebook, idx)
    loss = 1.25 * loss_sum[0, 0] / (B * N * D)
    return quant.reshape(z.shape), loss, idx.reshape(B, N)
